# baseline (speedup 1.0000x reference)
import jax
import jax.numpy as jnp
from jax import lax
from jax.experimental import pallas as pl
from jax.experimental.pallas import tpu as pltpu
from jax.experimental.pallas import tpu_sc as plsc

BATCH = 16384
D = 128
L = 16
NC = 2
NS = 16
NW = NC * NS

SC_ROWS = 8192
TC_ROWS = BATCH - SC_ROWS
ROWS_PER_W = SC_ROWS // NW
CHUNK = 128
NCHUNKS = ROWS_PER_W // CHUNK
TBLK = 1024


def _permute_body(tgt_hbm, perm_hbm, out_hbm, perm_v,
                  in0, in1, out0, out1,
                  sem_in0, sem_in1, sem_out0, sem_out1):
    wid = lax.axis_index("s") * NC + lax.axis_index("c")
    pltpu.sync_copy(perm_hbm, perm_v)
    pslices = [perm_v[pl.ds(j * L, L)] for j in range(D // L)]
    row0 = wid * ROWS_PER_W

    in_bufs = (in0, in1)
    out_bufs = (out0, out1)
    sem_in = (sem_in0, sem_in1)
    sem_out = (sem_out0, sem_out1)

    def rows(c):
        return pl.ds(row0 + c * CHUNK, CHUNK)

    def compute(in_ref, out_ref):
        @plsc.parallel_loop(0, CHUNK, unroll=4)
        def _(r):
            rvec = jnp.full((L,), 0, jnp.int32) + r
            for j in range(D // L):
                out_ref[r, pl.ds(j * L, L)] = plsc.load_gather(
                    in_ref, [rvec, pslices[j]])

    in_dma = [None] * NCHUNKS
    out_dma = [None] * NCHUNKS
    in_dma[0] = pltpu.async_copy(tgt_hbm.at[rows(0)], in_bufs[0], sem_in[0])
    for c in range(NCHUNKS):
        b = c % 2
        if c + 1 < NCHUNKS:
            in_dma[c + 1] = pltpu.async_copy(
                tgt_hbm.at[rows(c + 1)], in_bufs[1 - b], sem_in[1 - b])
        in_dma[c].wait()
        if c >= 2:
            out_dma[c - 2].wait()
        compute(in_bufs[b], out_bufs[b])
        out_dma[c] = pltpu.async_copy(out_bufs[b], out_hbm.at[rows(c)],
                                      sem_out[b])
    for c in range(max(0, NCHUNKS - 2), NCHUNKS):
        out_dma[c].wait()


def _tc_body(perm_ref, tgt_ref, out_ref):
    p = perm_ref[...]
    i = lax.broadcasted_iota(jnp.int32, (D, D), 0)
    onehot = (i == p).astype(jnp.float32)
    out_ref[...] = jnp.dot(tgt_ref[...], onehot,
                           preferred_element_type=jnp.float32)


def kernel(target, permutation):
    tc_out = pl.pallas_call(
        _tc_body,
        grid=(TC_ROWS // TBLK,),
        in_specs=[
            pl.BlockSpec((1, D), lambda i: (0, 0)),
            pl.BlockSpec((TBLK, D), lambda i: (i, 0)),
        ],
        out_specs=pl.BlockSpec((TBLK, D), lambda i: (SC_ROWS // TBLK + i, 0)),
        out_shape=jax.ShapeDtypeStruct((BATCH, D), jnp.float32),
    )(permutation.reshape(1, D), target[SC_ROWS:])

    mesh = plsc.VectorSubcoreMesh(core_axis_name="c", subcore_axis_name="s")
    sc_fill = pl.kernel(
        _permute_body,
        out_type=(),
        mesh=mesh,
        compiler_params=pltpu.CompilerParams(needs_layout_passes=False),
        scratch_types=[
            pltpu.VMEM((D,), jnp.int32),
            pltpu.VMEM((CHUNK, D), jnp.float32),
            pltpu.VMEM((CHUNK, D), jnp.float32),
            pltpu.VMEM((CHUNK, D), jnp.float32),
            pltpu.VMEM((CHUNK, D), jnp.float32),
            pltpu.SemaphoreType.DMA,
            pltpu.SemaphoreType.DMA,
            pltpu.SemaphoreType.DMA,
            pltpu.SemaphoreType.DMA,
        ],
    )
    buf = jax.new_ref(tc_out)
    sc_fill(target, permutation, buf)
    return buf[...]


# speedup vs baseline: 1.0886x; 1.0886x over previous
import jax
import jax.numpy as jnp
from jax import lax
from jax.experimental import pallas as pl
from jax.experimental.pallas import tpu as pltpu
from jax.experimental.pallas import tpu_sc as plsc

BATCH = 16384
D = 128
L = 16
NC = 2
NS = 16
NW = NC * NS

SC_ROWS = 8192
TC_ROWS = BATCH - SC_ROWS
ROWS_PER_W = SC_ROWS // NW
CHUNK = 128
NCHUNKS = ROWS_PER_W // CHUNK
TBLK = 1024


def _permute_body(tgt_hbm, perm_hbm, out_hbm, perm_v,
                  in0, in1, out0, out1,
                  sem_in0, sem_in1, sem_out0, sem_out1):
    wid = lax.axis_index("s") * NC + lax.axis_index("c")
    pltpu.sync_copy(perm_hbm, perm_v)
    pslices = [perm_v[pl.ds(j * L, L)] for j in range(D // L)]
    row0 = wid * ROWS_PER_W

    in_bufs = (in0, in1)
    out_bufs = (out0, out1)
    sem_in = (sem_in0, sem_in1)
    sem_out = (sem_out0, sem_out1)

    def rows(c):
        return pl.ds(row0 + c * CHUNK, CHUNK)

    def compute(in_ref, out_ref):
        @plsc.parallel_loop(0, CHUNK, unroll=4)
        def _(r):
            rvec = jnp.full((L,), 0, jnp.int32) + r
            for j in range(D // L):
                out_ref[r, pl.ds(j * L, L)] = plsc.load_gather(
                    in_ref, [rvec, pslices[j]])

    in_dma = [None] * NCHUNKS
    out_dma = [None] * NCHUNKS
    in_dma[0] = pltpu.async_copy(tgt_hbm.at[rows(0)], in_bufs[0], sem_in[0])
    for c in range(NCHUNKS):
        b = c % 2
        if c + 1 < NCHUNKS:
            in_dma[c + 1] = pltpu.async_copy(
                tgt_hbm.at[rows(c + 1)], in_bufs[1 - b], sem_in[1 - b])
        in_dma[c].wait()
        if c >= 2:
            out_dma[c - 2].wait()
        compute(in_bufs[b], out_bufs[b])
        out_dma[c] = pltpu.async_copy(out_bufs[b], out_hbm.at[rows(c)],
                                      sem_out[b])
    for c in range(max(0, NCHUNKS - 2), NCHUNKS):
        out_dma[c].wait()


def _tc_body(perm_ref, tgt_ref, out_ref):
    p = perm_ref[...]
    i = lax.broadcasted_iota(jnp.int32, (D, D), 0)
    onehot = (i == p).astype(jnp.float32)
    out_ref[...] = jnp.dot(tgt_ref[...], onehot,
                           preferred_element_type=jnp.float32,
                           precision=lax.Precision.HIGHEST)


def kernel(target, permutation):
    tc_out = pl.pallas_call(
        _tc_body,
        grid=(TC_ROWS // TBLK,),
        in_specs=[
            pl.BlockSpec((1, D), lambda i: (0, 0)),
            pl.BlockSpec((TBLK, D), lambda i: (SC_ROWS // TBLK + i, 0)),
        ],
        out_specs=pl.BlockSpec((TBLK, D), lambda i: (SC_ROWS // TBLK + i, 0)),
        out_shape=jax.ShapeDtypeStruct((BATCH, D), jnp.float32),
    )(permutation.reshape(1, D), target)

    mesh = plsc.VectorSubcoreMesh(core_axis_name="c", subcore_axis_name="s")
    sc_fill = pl.kernel(
        _permute_body,
        out_type=(),
        mesh=mesh,
        compiler_params=pltpu.CompilerParams(needs_layout_passes=False),
        scratch_types=[
            pltpu.VMEM((D,), jnp.int32),
            pltpu.VMEM((CHUNK, D), jnp.float32),
            pltpu.VMEM((CHUNK, D), jnp.float32),
            pltpu.VMEM((CHUNK, D), jnp.float32),
            pltpu.VMEM((CHUNK, D), jnp.float32),
            pltpu.SemaphoreType.DMA,
            pltpu.SemaphoreType.DMA,
            pltpu.SemaphoreType.DMA,
            pltpu.SemaphoreType.DMA,
        ],
    )
    buf = jax.new_ref(tc_out)
    sc_fill(target, permutation, buf)
    return buf[...]


# speedup vs baseline: 1.1923x; 1.0953x over previous
import jax
import jax.numpy as jnp
from jax import lax
from jax.experimental import pallas as pl
from jax.experimental.pallas import tpu as pltpu
from jax.experimental.pallas import tpu_sc as plsc

BATCH = 16384
D = 128
L = 16
NC = 2
NS = 16
NW = NC * NS
ROWS_PER_W = BATCH // NW
CHUNK = 128
NCHUNKS = ROWS_PER_W // CHUNK


def _permute_body(tgt_hbm, perm_hbm, out_hbm, perm_v,
                  in0, in1, out0, out1,
                  sem_in0, sem_in1, sem_out0, sem_out1):
    wid = lax.axis_index("s") * NC + lax.axis_index("c")
    pltpu.sync_copy(perm_hbm, perm_v)
    pslices = [perm_v[pl.ds(j * L, L)] for j in range(D // L)]
    row0 = wid * ROWS_PER_W

    in_bufs = (in0, in1)
    out_bufs = (out0, out1)
    sem_in = (sem_in0, sem_in1)
    sem_out = (sem_out0, sem_out1)

    def rows(c):
        return pl.ds(row0 + c * CHUNK, CHUNK)

    def compute(in_ref, out_ref):
        @plsc.parallel_loop(0, CHUNK, unroll=2)
        def _(r):
            rvec = jnp.full((L,), 0, jnp.int32) + r
            for j in range(D // L):
                out_ref[r, pl.ds(j * L, L)] = plsc.load_gather(
                    in_ref, [rvec, pslices[j]])

    in_dma = [None] * NCHUNKS
    out_dma = [None] * NCHUNKS
    in_dma[0] = pltpu.async_copy(tgt_hbm.at[rows(0)], in_bufs[0], sem_in[0])
    for c in range(NCHUNKS):
        b = c % 2
        if c + 1 < NCHUNKS:
            in_dma[c + 1] = pltpu.async_copy(
                tgt_hbm.at[rows(c + 1)], in_bufs[1 - b], sem_in[1 - b])
        in_dma[c].wait()
        if c >= 2:
            out_dma[c - 2].wait()
        compute(in_bufs[b], out_bufs[b])
        out_dma[c] = pltpu.async_copy(out_bufs[b], out_hbm.at[rows(c)],
                                      sem_out[b])
    for c in range(max(0, NCHUNKS - 2), NCHUNKS):
        out_dma[c].wait()


def kernel(target, permutation):
    mesh = plsc.VectorSubcoreMesh(core_axis_name="c", subcore_axis_name="s")
    k = pl.kernel(
        _permute_body,
        out_type=jax.ShapeDtypeStruct((BATCH, D), jnp.float32),
        mesh=mesh,
        compiler_params=pltpu.CompilerParams(needs_layout_passes=False),
        scratch_types=[
            pltpu.VMEM((D,), jnp.int32),
            pltpu.VMEM((CHUNK, D), jnp.float32),
            pltpu.VMEM((CHUNK, D), jnp.float32),
            pltpu.VMEM((CHUNK, D), jnp.float32),
            pltpu.VMEM((CHUNK, D), jnp.float32),
            pltpu.SemaphoreType.DMA,
            pltpu.SemaphoreType.DMA,
            pltpu.SemaphoreType.DMA,
            pltpu.SemaphoreType.DMA,
        ],
    )
    return k(target, permutation)


# speedup vs baseline: 1.2254x; 1.0277x over previous
import jax
import jax.numpy as jnp
from jax import lax
from jax.experimental import pallas as pl
from jax.experimental.pallas import tpu as pltpu
from jax.experimental.pallas import tpu_sc as plsc

BATCH = 16384
D = 128
L = 16
NC = 2
NS = 16
NW = NC * NS
ROWS_PER_W = BATCH // NW
CHUNK = 128
NCHUNKS = ROWS_PER_W // CHUNK


def _permute_body(tgt_hbm, perm_hbm, out_hbm, perm_v,
                  in0, in1, out0, out1,
                  sem_in0, sem_in1, sem_out0, sem_out1):
    wid = lax.axis_index("s") * NC + lax.axis_index("c")
    row0 = wid * ROWS_PER_W

    in_bufs = (in0, in1)
    out_bufs = (out0, out1)
    sem_in = (sem_in0, sem_in1)
    sem_out = (sem_out0, sem_out1)

    def rows(c):
        return pl.ds(row0 + c * CHUNK, CHUNK)

    in_dma = [None] * NCHUNKS
    out_dma = [None] * NCHUNKS
    in_dma[0] = pltpu.async_copy(tgt_hbm.at[rows(0)], in_bufs[0], sem_in[0])
    if NCHUNKS > 1:
        in_dma[1] = pltpu.async_copy(tgt_hbm.at[rows(1)], in_bufs[1], sem_in[1])

    pltpu.sync_copy(perm_hbm, perm_v)
    pslices = [perm_v[pl.ds(j * L, L)] for j in range(D // L)]

    def compute(in_ref, out_ref):
        @plsc.parallel_loop(0, CHUNK, unroll=2)
        def _(r):
            rvec = jnp.full((L,), 0, jnp.int32) + r
            for j in range(D // L):
                out_ref[r, pl.ds(j * L, L)] = plsc.load_gather(
                    in_ref, [rvec, pslices[j]])

    for c in range(NCHUNKS):
        b = c % 2
        in_dma[c].wait()
        if c >= 2:
            out_dma[c - 2].wait()
        compute(in_bufs[b], out_bufs[b])
        if c + 2 < NCHUNKS:
            in_dma[c + 2] = pltpu.async_copy(
                tgt_hbm.at[rows(c + 2)], in_bufs[b], sem_in[b])
        out_dma[c] = pltpu.async_copy(out_bufs[b], out_hbm.at[rows(c)],
                                      sem_out[b])
    for c in range(max(0, NCHUNKS - 2), NCHUNKS):
        out_dma[c].wait()


def kernel(target, permutation):
    mesh = plsc.VectorSubcoreMesh(core_axis_name="c", subcore_axis_name="s")
    k = pl.kernel(
        _permute_body,
        out_type=jax.ShapeDtypeStruct((BATCH, D), jnp.float32),
        mesh=mesh,
        compiler_params=pltpu.CompilerParams(needs_layout_passes=False),
        scratch_types=[
            pltpu.VMEM((D,), jnp.int32),
            pltpu.VMEM((CHUNK, D), jnp.float32),
            pltpu.VMEM((CHUNK, D), jnp.float32),
            pltpu.VMEM((CHUNK, D), jnp.float32),
            pltpu.VMEM((CHUNK, D), jnp.float32),
            pltpu.SemaphoreType.DMA,
            pltpu.SemaphoreType.DMA,
            pltpu.SemaphoreType.DMA,
            pltpu.SemaphoreType.DMA,
        ],
    )
    return k(target, permutation)
